# Initial kernel scaffold; baseline (speedup 1.0000x reference)
import jax
import jax.numpy as jnp
from jax.experimental import pallas as pl


def kernel(x1, x2, embedding):
    raise NotImplementedError("write your pallas kernel here")


# speedup vs baseline: 7.3485x; 7.3485x over previous
import functools

import jax
import jax.numpy as jnp
from jax import lax
from jax.experimental import pallas as pl
from jax.experimental.pallas import tpu as pltpu
from jax.experimental.pallas import tpu_sc as plsc

_VOCAB = 1_000_000
_DIM = 32
_B = 16384
_L = 200
_N = _B * _L
_NW = 32
_NPW = _N // _NW
_C = 512
_SUB = _C // 128
_NCHUNK = _NPW // _C
_ROWS_PER_W = _NPW // 128


def _rsqrt(x):
    i = lax.bitcast_convert_type(x, jnp.int32)
    y = lax.bitcast_convert_type(jnp.int32(0x5F3759DF) - (i >> 1),
                                 jnp.float32)
    half = jnp.float32(0.5)
    three_half = jnp.float32(1.5)
    for _ in range(3):
        y = y * (three_half - half * x * y * y)
    return y


_mesh = plsc.VectorSubcoreMesh(core_axis_name="c", subcore_axis_name="s")


@functools.partial(
    pl.kernel,
    mesh=_mesh,
    compiler_params=pltpu.CompilerParams(needs_layout_passes=False,
                                         use_tc_tiling_on_sc=False),
    out_type=jax.ShapeDtypeStruct((_N // 128, 128), jnp.float32),
    scratch_types=[
        pltpu.VMEM((_SUB, 128), jnp.int32),
        pltpu.VMEM((_SUB, 128), jnp.int32),
        pltpu.VMEM((_C, _DIM), jnp.float32),
        pltpu.VMEM((_C, _DIM), jnp.float32),
        pltpu.VMEM((_C * 17,), jnp.float32),
        pltpu.VMEM((_C * 17,), jnp.float32),
        pltpu.VMEM((_C * 17,), jnp.float32),
        pltpu.VMEM((_SUB, 128), jnp.float32),
        pltpu.SemaphoreType.DMA,
    ],
)
def _pairs_kernel(x1_hbm, x2_hbm, table_hbm, out_hbm,
                  idx1_v, idx2_v, rows1_v, rows2_v,
                  dot_s, n1_s, n2_s, out_v, sem):
    w = lax.axis_index("s") * 2 + lax.axis_index("c")
    row0 = w * _ROWS_PER_W

    def chunk_body(k, carry):
        rb = row0 + k * _SUB
        pltpu.sync_copy(x1_hbm.at[pl.ds(rb, _SUB)], idx1_v)
        pltpu.sync_copy(x2_hbm.at[pl.ds(rb, _SUB)], idx2_v)
        copies = []
        for j in range(_SUB):
            copies.append(pltpu.async_copy(
                table_hbm.at[idx1_v.at[j]],
                rows1_v.at[pl.ds(j * 128, 128)], sem))
            copies.append(pltpu.async_copy(
                table_hbm.at[idx2_v.at[j]],
                rows2_v.at[pl.ds(j * 128, 128)], sem))
        for cp in copies:
            cp.wait()

        def pair_body(i, acc):
            a1 = rows1_v[i, pl.ds(0, 16)]
            b1 = rows1_v[i, pl.ds(16, 16)]
            a2 = rows2_v[i, pl.ds(0, 16)]
            b2 = rows2_v[i, pl.ds(16, 16)]
            dot_s[pl.ds(i * 17, 16)] = plsc.cumsum(a1 * a2 + b1 * b2)
            n1_s[pl.ds(i * 17, 16)] = plsc.cumsum(a1 * a1 + b1 * b1)
            n2_s[pl.ds(i * 17, 16)] = plsc.cumsum(a2 * a2 + b2 * b2)
            return acc

        lax.fori_loop(0, _C, pair_body, 0, unroll=4)

        lane_ids = lax.iota(jnp.int32, 16)

        def grp_body(g, acc):
            ids = (g * 16 + lane_ids) * 17 + 15
            d = plsc.load_gather(dot_s, [ids])
            q1 = jnp.maximum(plsc.load_gather(n1_s, [ids]), jnp.float32(1.0))
            q2 = jnp.maximum(plsc.load_gather(n2_s, [ids]), jnp.float32(1.0))
            out_v[g // 8, pl.ds((g % 8) * 16, 16)] = d * _rsqrt(q1) * _rsqrt(q2)
            return acc

        lax.fori_loop(0, _C // 16, grp_body, 0)
        pltpu.sync_copy(out_v, out_hbm.at[pl.ds(rb, _SUB)])
        return carry

    lax.fori_loop(0, _NCHUNK, chunk_body, 0)


def kernel(x1, x2, embedding):
    x1f = x1.reshape(_N // 128, 128)
    x2f = x2.reshape(_N // 128, 128)
    out = _pairs_kernel(x1f, x2f, embedding)
    return out.reshape(_B, _L)


# speedup vs baseline: 15.7320x; 2.1408x over previous
import functools

import jax
import jax.numpy as jnp
from jax import lax
from jax.experimental import pallas as pl
from jax.experimental.pallas import tpu as pltpu
from jax.experimental.pallas import tpu_sc as plsc

_VOCAB = 1_000_000
_DIM = 32
_B = 16384
_L = 200
_N = _B * _L
_NW = 32
_NPW = _N // _NW
_C = 512
_SUB = _C // 128
_NCHUNK = _NPW // _C
_ROWS_PER_W = _NPW // 128


def _rsqrt(x):
    i = lax.bitcast_convert_type(x, jnp.int32)
    y = lax.bitcast_convert_type(jnp.int32(0x5F3759DF) - (i >> 1),
                                 jnp.float32)
    half = jnp.float32(0.5)
    three_half = jnp.float32(1.5)
    for _ in range(3):
        y = y * (three_half - half * x * y * y)
    return y


_mesh = plsc.VectorSubcoreMesh(core_axis_name="c", subcore_axis_name="s")


@functools.partial(
    pl.kernel,
    mesh=_mesh,
    compiler_params=pltpu.CompilerParams(needs_layout_passes=False,
                                         use_tc_tiling_on_sc=False),
    out_type=jax.ShapeDtypeStruct((_N // 128, 128), jnp.float32),
    scratch_types=[
        pltpu.VMEM((2, _SUB, 128), jnp.int32),
        pltpu.VMEM((2, _SUB, 128), jnp.int32),
        pltpu.VMEM((2, _C, _DIM), jnp.float32),
        pltpu.VMEM((2, _C, _DIM), jnp.float32),
        pltpu.VMEM((_C * 17,), jnp.float32),
        pltpu.VMEM((_C * 17,), jnp.float32),
        pltpu.VMEM((_C * 17,), jnp.float32),
        pltpu.VMEM((_SUB, 128), jnp.float32),
        pltpu.SemaphoreType.DMA((2,)),
    ],
)
def _pairs_kernel(x1_hbm, x2_hbm, table_hbm, out_hbm,
                  idx1_v, idx2_v, rows1_v, rows2_v,
                  dot_s, n1_s, n2_s, out_v, sems):
    w = lax.axis_index("s") * 2 + lax.axis_index("c")
    row0 = w * _ROWS_PER_W

    def issue(k, b):
        rb = row0 + k * _SUB
        pltpu.sync_copy(x1_hbm.at[pl.ds(rb, _SUB)], idx1_v.at[b])
        pltpu.sync_copy(x2_hbm.at[pl.ds(rb, _SUB)], idx2_v.at[b])
        for j in range(_SUB):
            pltpu.async_copy(table_hbm.at[idx1_v.at[b, j]],
                             rows1_v.at[b, pl.ds(j * 128, 128)], sems.at[b])
            pltpu.async_copy(table_hbm.at[idx2_v.at[b, j]],
                             rows2_v.at[b, pl.ds(j * 128, 128)], sems.at[b])

    def drain(b):
        pltpu.make_async_copy(table_hbm.at[pl.ds(0, _C)],
                              rows1_v.at[b], sems.at[b]).wait()
        pltpu.make_async_copy(table_hbm.at[pl.ds(0, _C)],
                              rows2_v.at[b], sems.at[b]).wait()

    lane_ids = lax.iota(jnp.int32, 16)

    def compute(k, b):
        @plsc.parallel_loop(0, _C, unroll=8)
        def pair_body(i):
            a1 = rows1_v[b, i, pl.ds(0, 16)]
            b1 = rows1_v[b, i, pl.ds(16, 16)]
            a2 = rows2_v[b, i, pl.ds(0, 16)]
            b2 = rows2_v[b, i, pl.ds(16, 16)]
            dot_s[pl.ds(i * 17, 16)] = plsc.cumsum(a1 * a2 + b1 * b2)
            n1_s[pl.ds(i * 17, 16)] = plsc.cumsum(a1 * a1 + b1 * b1)
            n2_s[pl.ds(i * 17, 16)] = plsc.cumsum(a2 * a2 + b2 * b2)

        @plsc.parallel_loop(0, _C // 16, unroll=4)
        def grp_body(g):
            ids = (g * 16 + lane_ids) * 17 + 15
            d = plsc.load_gather(dot_s, [ids])
            q1 = jnp.maximum(plsc.load_gather(n1_s, [ids]), jnp.float32(1.0))
            q2 = jnp.maximum(plsc.load_gather(n2_s, [ids]), jnp.float32(1.0))
            out_v[g // 8, pl.ds((g % 8) * 16, 16)] = (
                d * _rsqrt(q1) * _rsqrt(q2))

        pltpu.sync_copy(out_v, out_hbm.at[pl.ds(row0 + k * _SUB, _SUB)])

    issue(0, 0)

    def body(kk, carry):
        for b in range(2):
            k = kk * 2 + b
            issue(jnp.minimum(k + 1, _NCHUNK - 1), 1 - b)
            drain(b)
            compute(k, b)
        return carry

    lax.fori_loop(0, _NCHUNK // 2, body, 0)
    drain(0)


def kernel(x1, x2, embedding):
    x1f = x1.reshape(_N // 128, 128)
    x2f = x2.reshape(_N // 128, 128)
    out = _pairs_kernel(x1f, x2f, embedding)
    return out.reshape(_B, _L)
